# Initial kernel scaffold; baseline (speedup 1.0000x reference)
import jax
import jax.numpy as jnp
from jax.experimental import pallas as pl


def kernel(x, edge_index, edge_weight, node_lock, W, b):
    raise NotImplementedError("write your pallas kernel here")


# speedup vs baseline: 5.3589x; 5.3589x over previous
import functools

import jax
import jax.numpy as jnp
from jax import lax
from jax.experimental import pallas as pl
from jax.experimental.pallas import tpu as pltpu
from jax.experimental.pallas import tpu_sc as plsc

N = 10000
E = 320000
F = 128

NC = 2
NS = 16
L = 16

K = 128
NCHUNKS = E // K
NW = NC * NS
CHUNK_BASE = NCHUNKS // NW
CHUNK_REM = NCHUNKS % NW
ROWS_PER_TILE = 624


def _matmul_body(x_ref, wt_ref, o_ref):
    o_ref[...] = jnp.dot(x_ref[...], wt_ref[...],
                         preferred_element_type=jnp.float32)


def _combine_body(p_ref, b_ref, o_ref):
    o_ref[...] = p_ref[0] + p_ref[1] + b_ref[...]


def _sc_edge_body(h_hbm, src_hbm, dst_hbm, w_hbm, part_hbm,
                  acc_sh, src_v, dst_v, w_v, rows_v, sem):
    c = lax.axis_index("c")
    s = lax.axis_index("s")
    wid = c * NS + s

    zeros = jnp.zeros((L,), jnp.float32)

    @pl.loop(0, K)
    def _zero_rows(i):
        for t in range(F // L):
            rows_v[i, pl.ds(t * L, L)] = zeros

    row0 = s * ROWS_PER_TILE
    for off, nrow in ((0, 128), (128, 128), (256, 128), (384, 128), (512, 112)):
        pltpu.sync_copy(rows_v.at[pl.ds(0, nrow)],
                        acc_sh.at[pl.ds(row0 + off, nrow)])

    @pl.when(s == 0)
    def _zero_tail():
        pltpu.sync_copy(rows_v.at[pl.ds(0, 16)],
                        acc_sh.at[pl.ds(NS * ROWS_PER_TILE, 16)])

    plsc.subcore_barrier()

    cnt = CHUNK_BASE + jnp.where(wid < CHUNK_REM, 1, 0)
    start = wid * CHUNK_BASE + jnp.minimum(wid, CHUNK_REM)

    @pl.loop(0, cnt)
    def _chunk(j):
        base = (start + j) * K
        pltpu.sync_copy(src_hbm.at[pl.ds(base, K)], src_v)
        pltpu.sync_copy(dst_hbm.at[pl.ds(base, K)], dst_v)
        pltpu.sync_copy(w_hbm.at[pl.ds(base, K)], w_v)
        pltpu.async_copy(h_hbm.at[src_v], rows_v, sem).wait()

        @pl.loop(0, K // L)
        def _scale(g):
            wv = w_v[pl.ds(g * L, L)]
            for l in range(L):
                wb = lax.gather(
                    wv, jnp.full((L, 1), l, jnp.int32),
                    lax.GatherDimensionNumbers(offset_dims=(),
                                               collapsed_slice_dims=(0,),
                                               start_index_map=(0,)),
                    (1,), mode=lax.GatherScatterMode.PROMISE_IN_BOUNDS)
                e = g * L + l
                for t in range(F // L):
                    rows_v[e, pl.ds(t * L, L)] = rows_v[e, pl.ds(t * L, L)] * wb

        pltpu.sync_copy(rows_v, acc_sh.at[dst_v], add=True)

    plsc.subcore_barrier()
    pltpu.sync_copy(acc_sh.at[pl.ds(row0, ROWS_PER_TILE)],
                    part_hbm.at[c, pl.ds(row0, ROWS_PER_TILE)])

    @pl.when(s == 0)
    def _flush_tail():
        pltpu.sync_copy(acc_sh.at[pl.ds(NS * ROWS_PER_TILE, 16)],
                        part_hbm.at[c, pl.ds(NS * ROWS_PER_TILE, 16)])


def kernel(x, edge_index, edge_weight, node_lock, W, b):
    del node_lock
    h = pl.pallas_call(
        _matmul_body,
        grid=(10,),
        in_specs=[pl.BlockSpec((N // 10, F), lambda i: (i, 0)),
                  pl.BlockSpec((F, F), lambda i: (0, 0))],
        out_specs=pl.BlockSpec((N // 10, F), lambda i: (i, 0)),
        out_shape=jax.ShapeDtypeStruct((N, F), jnp.float32),
    )(x, W.T)

    mesh = plsc.VectorSubcoreMesh(core_axis_name="c", subcore_axis_name="s",
                                  num_cores=NC, num_subcores=NS)
    sc_edges = pl.kernel(
        _sc_edge_body,
        out_type=jax.ShapeDtypeStruct((NC, N, F), jnp.float32),
        mesh=mesh,
        scratch_types=[
            pltpu.VMEM_SHARED((N, F), jnp.float32),
            pltpu.VMEM((K,), jnp.int32),
            pltpu.VMEM((K,), jnp.int32),
            pltpu.VMEM((K,), jnp.float32),
            pltpu.VMEM((K, F), jnp.float32),
            pltpu.SemaphoreType.DMA,
        ],
    )
    part = sc_edges(h, edge_index[0], edge_index[1], edge_weight)

    out = pl.pallas_call(
        _combine_body,
        grid=(10,),
        in_specs=[pl.BlockSpec((NC, N // 10, F), lambda i: (0, i, 0)),
                  pl.BlockSpec((1, F), lambda i: (0, 0))],
        out_specs=pl.BlockSpec((N // 10, F), lambda i: (i, 0)),
        out_shape=jax.ShapeDtypeStruct((N, F), jnp.float32),
    )(part, b.reshape(1, F))

    return (out, edge_index, edge_weight)


# speedup vs baseline: 7.3005x; 1.3623x over previous
import functools

import jax
import jax.numpy as jnp
from jax import lax
from jax.experimental import pallas as pl
from jax.experimental.pallas import tpu as pltpu
from jax.experimental.pallas import tpu_sc as plsc

N = 10000
E = 320000
F = 128

NC = 2
NS = 16
L = 16

K = 128
NW = NC * NS
EDGES_PER_TILE = E // NW
NCH = EDGES_PER_TILE // K
TAIL = EDGES_PER_TILE - NCH * K
ROWS_PER_TILE = 624


def _matmul_body(x_ref, wt_ref, o_ref):
    o_ref[...] = jnp.dot(x_ref[...], wt_ref[...],
                         preferred_element_type=jnp.float32)


def _combine_body(p_ref, b_ref, o_ref):
    o_ref[...] = p_ref[0] + p_ref[1] + b_ref[...]


def _lane_broadcast(wv, l):
    return lax.gather(
        wv, jnp.full((L, 1), l, jnp.int32),
        lax.GatherDimensionNumbers(offset_dims=(),
                                   collapsed_slice_dims=(0,),
                                   start_index_map=(0,)),
        (1,), mode=lax.GatherScatterMode.PROMISE_IN_BOUNDS)


def _sc_edge_body(h_hbm, src_hbm, dst_hbm, w_hbm, part_hbm,
                  acc_sh, src_v, dst_v, w_v, rows_v,
                  src_t, dst_t, w_t, sem0, sem1):
    c = lax.axis_index("c")
    s = lax.axis_index("s")
    wid = c * NS + s
    sems = (sem0, sem1)

    zeros = jnp.zeros((L,), jnp.float32)

    @pl.loop(0, K)
    def _zero_rows(i):
        for t in range(F // L):
            rows_v[0, i, pl.ds(t * L, L)] = zeros
            rows_v[1, i, pl.ds(t * L, L)] = zeros

    row0 = s * ROWS_PER_TILE
    for off, nrow in ((0, 128), (128, 128), (256, 128), (384, 128), (512, 112)):
        pltpu.sync_copy(rows_v.at[0].at[pl.ds(0, nrow)],
                        acc_sh.at[pl.ds(row0 + off, nrow)])

    @pl.when(s == 0)
    def _zero_tail():
        pltpu.sync_copy(rows_v.at[0].at[pl.ds(0, 16)],
                        acc_sh.at[pl.ds(NS * ROWS_PER_TILE, 16)])

    plsc.subcore_barrier()

    tbase = wid * EDGES_PER_TILE

    def load_and_gather(j, b):
        base = tbase + j * K
        pltpu.sync_copy(src_hbm.at[pl.ds(base, K)], src_v.at[b])
        pltpu.sync_copy(dst_hbm.at[pl.ds(base, K)], dst_v.at[b])
        pltpu.sync_copy(w_hbm.at[pl.ds(base, K)], w_v.at[b])
        pltpu.async_copy(h_hbm.at[src_v.at[b]], rows_v.at[b], sems[b])

    def scale_and_scatter(b):
        pltpu.make_async_copy(h_hbm.at[src_v.at[b]], rows_v.at[b],
                              sems[b]).wait()

        @pl.loop(0, K // L)
        def _scale(g):
            wv = w_v[b, pl.ds(g * L, L)]
            for l in range(L):
                wb = _lane_broadcast(wv, l)
                e = g * L + l
                for t in range(F // L):
                    rows_v[b, e, pl.ds(t * L, L)] = (
                        rows_v[b, e, pl.ds(t * L, L)] * wb)

        pltpu.sync_copy(rows_v.at[b], acc_sh.at[dst_v.at[b]], add=True)

    load_and_gather(0, 0)
    load_and_gather(1, 1)

    @pl.loop(0, NCH // 2)
    def _steps(m):
        j0 = m * 2
        scale_and_scatter(0)

        @pl.when(j0 + 2 < NCH)
        def _():
            load_and_gather(j0 + 2, 0)

        scale_and_scatter(1)

        @pl.when(j0 + 3 < NCH)
        def _():
            load_and_gather(j0 + 3, 1)

    base = tbase + NCH * K
    pltpu.sync_copy(src_hbm.at[pl.ds(base, TAIL)], src_t)
    pltpu.sync_copy(dst_hbm.at[pl.ds(base, TAIL)], dst_t)
    pltpu.sync_copy(w_hbm.at[pl.ds(base, TAIL)], w_t)
    pltpu.async_copy(h_hbm.at[src_t], rows_v.at[0].at[pl.ds(0, TAIL)],
                     sem0).wait()
    wv = w_t[...]
    for l in range(TAIL):
        wb = _lane_broadcast(wv, l)
        for t in range(F // L):
            rows_v[0, l, pl.ds(t * L, L)] = rows_v[0, l, pl.ds(t * L, L)] * wb
    pltpu.sync_copy(rows_v.at[0].at[pl.ds(0, TAIL)], acc_sh.at[dst_t],
                    add=True)

    plsc.subcore_barrier()
    pltpu.sync_copy(acc_sh.at[pl.ds(row0, ROWS_PER_TILE)],
                    part_hbm.at[c, pl.ds(row0, ROWS_PER_TILE)])

    @pl.when(s == 0)
    def _flush_tail():
        pltpu.sync_copy(acc_sh.at[pl.ds(NS * ROWS_PER_TILE, 16)],
                        part_hbm.at[c, pl.ds(NS * ROWS_PER_TILE, 16)])


def kernel(x, edge_index, edge_weight, node_lock, W, b):
    del node_lock
    h = pl.pallas_call(
        _matmul_body,
        grid=(10,),
        in_specs=[pl.BlockSpec((N // 10, F), lambda i: (i, 0)),
                  pl.BlockSpec((F, F), lambda i: (0, 0))],
        out_specs=pl.BlockSpec((N // 10, F), lambda i: (i, 0)),
        out_shape=jax.ShapeDtypeStruct((N, F), jnp.float32),
    )(x, W.T)

    mesh = plsc.VectorSubcoreMesh(core_axis_name="c", subcore_axis_name="s",
                                  num_cores=NC, num_subcores=NS)
    sc_edges = pl.kernel(
        _sc_edge_body,
        out_type=jax.ShapeDtypeStruct((NC, N, F), jnp.float32),
        mesh=mesh,
        scratch_types=[
            pltpu.VMEM_SHARED((N, F), jnp.float32),
            pltpu.VMEM((2, K), jnp.int32),
            pltpu.VMEM((2, K), jnp.int32),
            pltpu.VMEM((2, K), jnp.float32),
            pltpu.VMEM((2, K, F), jnp.float32),
            pltpu.VMEM((TAIL,), jnp.int32),
            pltpu.VMEM((TAIL,), jnp.int32),
            pltpu.VMEM((TAIL,), jnp.float32),
            pltpu.SemaphoreType.DMA,
            pltpu.SemaphoreType.DMA,
        ],
    )
    part = sc_edges(h, edge_index[0], edge_index[1], edge_weight)

    out = pl.pallas_call(
        _combine_body,
        grid=(10,),
        in_specs=[pl.BlockSpec((NC, N // 10, F), lambda i: (0, i, 0)),
                  pl.BlockSpec((1, F), lambda i: (0, 0))],
        out_specs=pl.BlockSpec((N // 10, F), lambda i: (i, 0)),
        out_shape=jax.ShapeDtypeStruct((N, F), jnp.float32),
    )(part, b.reshape(1, F))

    return (out, edge_index, edge_weight)
